# baseline (speedup 1.0000x reference)
import functools

import jax
import jax.numpy as jnp
from jax import lax
from jax.experimental import pallas as pl
from jax.experimental.pallas import tpu as pltpu
from jax.experimental.pallas import tpu_sc as plsc

B, S, H = 1, 2048, 1024
E, TOPK = 8, 2
I_MOE = 512
EPS = 1e-06
T = B * S
CAP = (T * TOPK // E) * 2
TRASH = E * CAP
BUF_ROWS = E * CAP + CAP
TB = 256
BLK = 512
NW = 32
CHT = T // NW


PK = H // 2


def _pack2(x):
    a = x[:, :PK].astype(jnp.bfloat16).astype(jnp.float32)
    b = x[:, PK:].astype(jnp.bfloat16).astype(jnp.float32)
    ai = lax.shift_right_logical(lax.bitcast_convert_type(a, jnp.int32), 16)
    bi = lax.bitwise_and(lax.bitcast_convert_type(b, jnp.int32),
                         jnp.int32(-65536))
    return lax.bitwise_or(ai, bi)


def _unpack2(w):
    a = lax.bitcast_convert_type(lax.shift_left(w, 16), jnp.float32)
    b = lax.bitcast_convert_type(lax.bitwise_and(w, jnp.int32(-65536)),
                                 jnp.float32)
    return jnp.concatenate([a, b], axis=1).astype(jnp.bfloat16)


def _router_body(hid_ref, lnw_ref, gw_ref,
                 xn_ref, wk0_ref, wk1_ref, cidx0_ref, cidx1_ref,
                 sidx0_ref, sidx1_ref, cnt_ref, carry_ref):
    pid = pl.program_id(0)

    @pl.when(pid == 0)
    def _():
        carry_ref[...] = jnp.zeros((1, E), jnp.float32)

    h = hid_ref[...]
    var = jnp.mean(h * h, axis=1, keepdims=True)
    xn = h * lax.rsqrt(var + EPS) * lnw_ref[...]
    xn_ref[...] = _pack2(xn)

    logits = lax.dot_general(xn, gw_ref[...], (((1,), (1,)), ((), ())),
                             preferred_element_type=jnp.float32)
    m = jnp.max(logits, axis=1, keepdims=True)
    ex = jnp.exp(logits - m)
    p = ex / jnp.sum(ex, axis=1, keepdims=True)

    j = lax.broadcasted_iota(jnp.int32, (TB, E), 1)
    m1 = jnp.max(p, axis=1, keepdims=True)
    i1 = jnp.min(jnp.where(p == m1, j, E), axis=1, keepdims=True)
    oh1 = j == i1
    p2 = jnp.where(oh1, -1.0, p)
    m2 = jnp.max(p2, axis=1, keepdims=True)
    i2 = jnp.min(jnp.where(p2 == m2, j, E), axis=1, keepdims=True)
    oh2 = j == i2
    w1 = jnp.sum(jnp.where(oh1, p, 0.0), axis=1, keepdims=True)
    w2 = jnp.sum(jnp.where(oh2, p, 0.0), axis=1, keepdims=True)
    s = w1 + w2 + 1e-20
    w1 = w1 / s
    w2 = w2 / s

    oh1f = oh1.astype(jnp.float32)
    oh2f = oh2.astype(jnp.float32)
    r = lax.broadcasted_iota(jnp.int32, (TB, TB), 0)
    c = lax.broadcasted_iota(jnp.int32, (TB, TB), 1)
    tri = (c <= r).astype(jnp.float32)
    cum1 = jnp.dot(tri, oh1f, preferred_element_type=jnp.float32)
    cum2 = jnp.dot(tri, oh2f, preferred_element_type=jnp.float32)
    mx = carry_ref[...] + cum1 + cum2 - oh1f - oh2f
    pos1 = jnp.sum(jnp.where(oh1, mx, 0.0), axis=1, keepdims=True).astype(jnp.int32)
    pos2 = jnp.sum(jnp.where(oh2, mx, 0.0), axis=1, keepdims=True).astype(jnp.int32)
    newc = carry_ref[...] + jnp.sum(oh1f + oh2f, axis=0, keepdims=True)
    carry_ref[...] = newc
    cnt_ref[...] = newc.astype(jnp.int32)

    slot1 = i1 * CAP + pos1
    slot2 = i2 * CAP + pos2
    keep1 = pos1 < CAP
    keep2 = pos2 < CAP
    cidx0_ref[...] = jnp.where(keep1, slot1, i1 * CAP)
    cidx1_ref[...] = jnp.where(keep2, slot2, i2 * CAP)
    sidx0_ref[...] = jnp.where(keep1, slot1, TRASH)
    sidx1_ref[...] = jnp.where(keep2, slot2, TRASH)
    wk0_ref[...] = jnp.where(keep1, w1, 0.0)
    wk1_ref[...] = jnp.where(keep2, w2, 0.0)


def _router(x, gate_weight, ln_weight):
    n = T // TB
    col = jax.ShapeDtypeStruct((T, 1), jnp.float32)
    coli = jax.ShapeDtypeStruct((T, 1), jnp.int32)
    return pl.pallas_call(
        _router_body,
        grid=(n,),
        in_specs=[
            pl.BlockSpec((TB, H), lambda i: (i, 0)),
            pl.BlockSpec((1, H), lambda i: (0, 0)),
            pl.BlockSpec((E, H), lambda i: (0, 0)),
        ],
        out_specs=[
            pl.BlockSpec((TB, PK), lambda i: (i, 0)),
            pl.BlockSpec((TB, 1), lambda i: (i, 0)),
            pl.BlockSpec((TB, 1), lambda i: (i, 0)),
            pl.BlockSpec((TB, 1), lambda i: (i, 0)),
            pl.BlockSpec((TB, 1), lambda i: (i, 0)),
            pl.BlockSpec((TB, 1), lambda i: (i, 0)),
            pl.BlockSpec((TB, 1), lambda i: (i, 0)),
            pl.BlockSpec((1, E), lambda i: (0, 0)),
        ],
        out_shape=[
            jax.ShapeDtypeStruct((T, PK), jnp.int32),
            col, col, coli, coli, coli, coli,
            jax.ShapeDtypeStruct((1, E), jnp.int32),
        ],
        scratch_shapes=[pltpu.VMEM((1, E), jnp.float32)],
    )(x, ln_weight.reshape(1, H), gate_weight)


def _moe_scatter(xn, sidx0, sidx1):
    mesh = plsc.VectorSubcoreMesh(core_axis_name="c", subcore_axis_name="s")

    @functools.partial(
        pl.kernel,
        mesh=mesh,
        out_type=jax.ShapeDtypeStruct((BUF_ROWS, PK), jnp.int32),
        scratch_types=[
            pltpu.VMEM((CHT,), jnp.int32),
            pltpu.VMEM((CHT, PK), jnp.int32),
            pltpu.SemaphoreType.DMA,
        ],
    )
    def scat(xn_hbm, s0_hbm, s1_hbm, buf_hbm, idx_v, rows_v, sem):
        wid = lax.axis_index("s") * 2 + lax.axis_index("c")
        base = wid * CHT
        pltpu.sync_copy(xn_hbm.at[pl.ds(base, CHT)], rows_v)
        pltpu.sync_copy(s0_hbm.at[pl.ds(base, CHT)], idx_v)
        pltpu.async_copy(rows_v, buf_hbm.at[idx_v], sem).wait()
        pltpu.sync_copy(s1_hbm.at[pl.ds(base, CHT)], idx_v)
        pltpu.async_copy(rows_v, buf_hbm.at[idx_v], sem).wait()

    return scat(xn, sidx0, sidx1)


FB = CAP // 2


def _ffn_body(cnt_ref, buf_ref, w13_ref, w2_ref, eo_ref, w13b_ref, w2b_ref):
    e = pl.program_id(0)
    c = pl.program_id(1)
    cnt = cnt_ref[0, e]

    @pl.when(c == 0)
    def _():
        w13b_ref[...] = w13_ref[0].astype(jnp.bfloat16)
        w2b_ref[...] = w2_ref[0].astype(jnp.bfloat16)

    @pl.when((c == 0) | (c * FB < cnt))
    def _():
        ridx = lax.broadcasted_iota(jnp.int32, (FB, 1), 0) + c * FB
        x = jnp.where(ridx < cnt, _unpack2(buf_ref[...]), jnp.bfloat16(0))
        gu = lax.dot_general(x, w13b_ref[...], (((1,), (0,)), ((), ())),
                             preferred_element_type=jnp.float32)
        g = gu[:, :I_MOE]
        u = gu[:, I_MOE:]
        a = (g * lax.logistic(g) * u).astype(jnp.bfloat16)
        eo_ref[...] = _pack2(
            lax.dot_general(a, w2b_ref[...], (((1,), (0,)), ((), ())),
                            preferred_element_type=jnp.float32))


def _expert_ffn(cnt, buf, w1w3, w2):
    nc = CAP // FB
    return pl.pallas_call(
        _ffn_body,
        grid=(E, nc),
        in_specs=[
            pl.BlockSpec(memory_space=pltpu.SMEM),
            pl.BlockSpec((FB, PK), lambda e, c: (e * nc + c, 0)),
            pl.BlockSpec((1, H, 2 * I_MOE), lambda e, c: (e, 0, 0)),
            pl.BlockSpec((1, I_MOE, H), lambda e, c: (e, 0, 0)),
        ],
        out_specs=pl.BlockSpec((FB, PK), lambda e, c: (e * nc + c, 0)),
        out_shape=jax.ShapeDtypeStruct((E * CAP, PK), jnp.int32),
        scratch_shapes=[
            pltpu.VMEM((H, 2 * I_MOE), jnp.bfloat16),
            pltpu.VMEM((I_MOE, H), jnp.bfloat16),
        ],
    )(cnt, buf, w1w3, w2)


def _combine_gather(eo, cidx0, cidx1):
    mesh = plsc.VectorSubcoreMesh(core_axis_name="c", subcore_axis_name="s")

    @functools.partial(
        pl.kernel,
        mesh=mesh,
        out_type=(
            jax.ShapeDtypeStruct((T, PK), jnp.int32),
            jax.ShapeDtypeStruct((T, PK), jnp.int32),
        ),
        scratch_types=[
            pltpu.VMEM((CHT,), jnp.int32),
            pltpu.VMEM((CHT, PK), jnp.int32),
            pltpu.SemaphoreType.DMA,
        ],
    )
    def gath(eo_hbm, c0_hbm, c1_hbm, g0_hbm, g1_hbm, idx_v, rows_v, sem):
        wid = lax.axis_index("s") * 2 + lax.axis_index("c")
        base = wid * CHT
        pltpu.sync_copy(c0_hbm.at[pl.ds(base, CHT)], idx_v)
        pltpu.async_copy(eo_hbm.at[idx_v], rows_v, sem).wait()
        pltpu.sync_copy(rows_v, g0_hbm.at[pl.ds(base, CHT)])
        pltpu.sync_copy(c1_hbm.at[pl.ds(base, CHT)], idx_v)
        pltpu.async_copy(eo_hbm.at[idx_v], rows_v, sem).wait()
        pltpu.sync_copy(rows_v, g1_hbm.at[pl.ds(base, CHT)])

    return gath(eo, cidx0, cidx1)


def _shared_body(xn_ref, sg_ref, su_ref, sd_ref, sh_ref,
                 sgb_ref, sub_ref, sdb_ref):
    @pl.when(pl.program_id(0) == 0)
    def _():
        sgb_ref[...] = sg_ref[...].astype(jnp.bfloat16)
        sub_ref[...] = su_ref[...].astype(jnp.bfloat16)
        sdb_ref[...] = sd_ref[...].astype(jnp.bfloat16)

    SB = 512
    for s in range(1024 // SB):
        xb = _unpack2(xn_ref[pl.ds(s * SB, SB), :])
        g = lax.dot_general(xb, sgb_ref[...], (((1,), (0,)), ((), ())),
                            preferred_element_type=jnp.float32)
        u = lax.dot_general(xb, sub_ref[...], (((1,), (0,)), ((), ())),
                            preferred_element_type=jnp.float32)
        a = (g * lax.logistic(g) * u).astype(jnp.bfloat16)
        sh_ref[pl.ds(s * SB, SB), :] = lax.dot_general(
            a, sdb_ref[...], (((1,), (0,)), ((), ())),
            preferred_element_type=jnp.float32).astype(jnp.bfloat16)


def _shared_ffn(xn, sg, su, sd):
    i_sh = sg.shape[1]
    return pl.pallas_call(
        _shared_body,
        grid=(2,),
        in_specs=[
            pl.BlockSpec((1024, PK), lambda i: (i, 0)),
            pl.BlockSpec((H, i_sh), lambda i: (0, 0)),
            pl.BlockSpec((H, i_sh), lambda i: (0, 0)),
            pl.BlockSpec((i_sh, H), lambda i: (0, 0)),
        ],
        out_specs=pl.BlockSpec((1024, H), lambda i: (i, 0)),
        out_shape=jax.ShapeDtypeStruct((T, H), jnp.bfloat16),
        scratch_shapes=[
            pltpu.VMEM((H, i_sh), jnp.bfloat16),
            pltpu.VMEM((H, i_sh), jnp.bfloat16),
            pltpu.VMEM((i_sh, H), jnp.bfloat16),
        ],
    )(xn, sg, su, sd)


def _combine_body(sh_ref, g0_ref, g1_ref, wk0_ref, wk1_ref, hid_ref, out_ref):
    out_ref[...] = (sh_ref[...].astype(jnp.float32)
                    + _unpack2(g0_ref[...]).astype(jnp.float32) * wk0_ref[...]
                    + _unpack2(g1_ref[...]).astype(jnp.float32) * wk1_ref[...]
                    + hid_ref[...])


def _combine(sh, g0, g1, wk0, wk1, hid):
    n = T // TB
    return pl.pallas_call(
        _combine_body,
        grid=(n,),
        in_specs=[
            pl.BlockSpec((TB, H), lambda i: (i, 0)),
            pl.BlockSpec((TB, PK), lambda i: (i, 0)),
            pl.BlockSpec((TB, PK), lambda i: (i, 0)),
            pl.BlockSpec((TB, 1), lambda i: (i, 0)),
            pl.BlockSpec((TB, 1), lambda i: (i, 0)),
            pl.BlockSpec((TB, H), lambda i: (i, 0)),
        ],
        out_specs=pl.BlockSpec((TB, H), lambda i: (i, 0)),
        out_shape=jax.ShapeDtypeStruct((T, H), jnp.float32),
    )(sh, g0, g1, wk0, wk1, hid)


def kernel(hidden_states, gate_weight, w1w3, w2, shared_gate, shared_up,
           shared_down, ln_weight):
    x = hidden_states.reshape(T, H)
    (xn, wk0, wk1, cidx0, cidx1, sidx0, sidx1, cnt) = _router(
        x, gate_weight, ln_weight)
    sh = _shared_ffn(xn, shared_gate, shared_up, shared_down)
    buf = _moe_scatter(xn, sidx0.reshape(T), sidx1.reshape(T))
    eo = _expert_ffn(cnt, buf, w1w3, w2)
    g0, g1 = _combine_gather(eo, cidx0.reshape(T), cidx1.reshape(T))
    out = _combine(sh, g0, g1, wk0, wk1, x)
    return out.reshape(B, S, H)


# speedup vs baseline: 1.1545x; 1.1545x over previous
import functools

import jax
import jax.numpy as jnp
from jax import lax
from jax.experimental import pallas as pl
from jax.experimental.pallas import tpu as pltpu
from jax.experimental.pallas import tpu_sc as plsc

B, S, H = 1, 2048, 1024
E, TOPK = 8, 2
I_MOE = 512
EPS = 1e-06
T = B * S
CAP = (T * TOPK // E) * 2
TRASH = E * CAP
BUF_ROWS = E * CAP + CAP
TB = 256
BLK = 256
NW = 32
CHT = T // NW


PK = H // 2


def _pack2(x):
    a = x[:, :PK].astype(jnp.bfloat16).astype(jnp.float32)
    b = x[:, PK:].astype(jnp.bfloat16).astype(jnp.float32)
    ai = lax.shift_right_logical(lax.bitcast_convert_type(a, jnp.int32), 16)
    bi = lax.bitwise_and(lax.bitcast_convert_type(b, jnp.int32),
                         jnp.int32(-65536))
    return lax.bitwise_or(ai, bi)


def _unpack2(w):
    a = lax.bitcast_convert_type(lax.shift_left(w, 16), jnp.float32)
    b = lax.bitcast_convert_type(lax.bitwise_and(w, jnp.int32(-65536)),
                                 jnp.float32)
    return jnp.concatenate([a, b], axis=1).astype(jnp.bfloat16)


def _router_body(hid_ref, lnw_ref, gw_ref,
                 xn_ref, wk0_ref, wk1_ref, cidx0_ref, cidx1_ref,
                 sidx0_ref, sidx1_ref, cnt_ref, carry_ref):
    pid = pl.program_id(0)

    @pl.when(pid == 0)
    def _():
        carry_ref[...] = jnp.zeros((1, E), jnp.float32)

    h = hid_ref[...]
    var = jnp.mean(h * h, axis=1, keepdims=True)
    xn = h * lax.rsqrt(var + EPS) * lnw_ref[...]
    xn_ref[...] = _pack2(xn)

    logits = lax.dot_general(xn, gw_ref[...], (((1,), (1,)), ((), ())),
                             preferred_element_type=jnp.float32)
    m = jnp.max(logits, axis=1, keepdims=True)
    ex = jnp.exp(logits - m)
    p = ex / jnp.sum(ex, axis=1, keepdims=True)

    j = lax.broadcasted_iota(jnp.int32, (TB, E), 1)
    m1 = jnp.max(p, axis=1, keepdims=True)
    i1 = jnp.min(jnp.where(p == m1, j, E), axis=1, keepdims=True)
    oh1 = j == i1
    p2 = jnp.where(oh1, -1.0, p)
    m2 = jnp.max(p2, axis=1, keepdims=True)
    i2 = jnp.min(jnp.where(p2 == m2, j, E), axis=1, keepdims=True)
    oh2 = j == i2
    w1 = jnp.sum(jnp.where(oh1, p, 0.0), axis=1, keepdims=True)
    w2 = jnp.sum(jnp.where(oh2, p, 0.0), axis=1, keepdims=True)
    s = w1 + w2 + 1e-20
    w1 = w1 / s
    w2 = w2 / s

    oh1f = oh1.astype(jnp.float32)
    oh2f = oh2.astype(jnp.float32)
    r = lax.broadcasted_iota(jnp.int32, (TB, TB), 0)
    c = lax.broadcasted_iota(jnp.int32, (TB, TB), 1)
    tri = (c <= r).astype(jnp.float32)
    cum1 = jnp.dot(tri, oh1f, preferred_element_type=jnp.float32)
    cum2 = jnp.dot(tri, oh2f, preferred_element_type=jnp.float32)
    mx = carry_ref[...] + cum1 + cum2 - oh1f - oh2f
    pos1 = jnp.sum(jnp.where(oh1, mx, 0.0), axis=1, keepdims=True).astype(jnp.int32)
    pos2 = jnp.sum(jnp.where(oh2, mx, 0.0), axis=1, keepdims=True).astype(jnp.int32)
    newc = carry_ref[...] + jnp.sum(oh1f + oh2f, axis=0, keepdims=True)
    carry_ref[...] = newc
    cnt_ref[...] = newc.astype(jnp.int32)

    slot1 = i1 * CAP + pos1
    slot2 = i2 * CAP + pos2
    keep1 = pos1 < CAP
    keep2 = pos2 < CAP
    cidx0_ref[...] = jnp.where(keep1, slot1, i1 * CAP)
    cidx1_ref[...] = jnp.where(keep2, slot2, i2 * CAP)
    sidx0_ref[...] = jnp.where(keep1, slot1, TRASH)
    sidx1_ref[...] = jnp.where(keep2, slot2, TRASH)
    wk0_ref[...] = jnp.where(keep1, w1, 0.0)
    wk1_ref[...] = jnp.where(keep2, w2, 0.0)


def _router(x, gate_weight, ln_weight):
    n = T // TB
    col = jax.ShapeDtypeStruct((T, 1), jnp.float32)
    coli = jax.ShapeDtypeStruct((T, 1), jnp.int32)
    return pl.pallas_call(
        _router_body,
        grid=(n,),
        in_specs=[
            pl.BlockSpec((TB, H), lambda i: (i, 0)),
            pl.BlockSpec((1, H), lambda i: (0, 0)),
            pl.BlockSpec((E, H), lambda i: (0, 0)),
        ],
        out_specs=[
            pl.BlockSpec((TB, PK), lambda i: (i, 0)),
            pl.BlockSpec((TB, 1), lambda i: (i, 0)),
            pl.BlockSpec((TB, 1), lambda i: (i, 0)),
            pl.BlockSpec((TB, 1), lambda i: (i, 0)),
            pl.BlockSpec((TB, 1), lambda i: (i, 0)),
            pl.BlockSpec((TB, 1), lambda i: (i, 0)),
            pl.BlockSpec((TB, 1), lambda i: (i, 0)),
            pl.BlockSpec((1, E), lambda i: (0, 0)),
        ],
        out_shape=[
            jax.ShapeDtypeStruct((T, PK), jnp.int32),
            col, col, coli, coli, coli, coli,
            jax.ShapeDtypeStruct((1, E), jnp.int32),
        ],
        scratch_shapes=[pltpu.VMEM((1, E), jnp.float32)],
    )(x, ln_weight.reshape(1, H), gate_weight)


def _moe_scatter(xn, sidx0, sidx1):
    mesh = plsc.VectorSubcoreMesh(core_axis_name="c", subcore_axis_name="s")

    @functools.partial(
        pl.kernel,
        mesh=mesh,
        out_type=jax.ShapeDtypeStruct((BUF_ROWS, PK), jnp.int32),
        scratch_types=[
            pltpu.VMEM((CHT,), jnp.int32),
            pltpu.VMEM((CHT, PK), jnp.int32),
            pltpu.SemaphoreType.DMA,
        ],
    )
    def scat(xn_hbm, s0_hbm, s1_hbm, buf_hbm, idx_v, rows_v, sem):
        wid = lax.axis_index("s") * 2 + lax.axis_index("c")
        base = wid * CHT
        pltpu.sync_copy(xn_hbm.at[pl.ds(base, CHT)], rows_v)
        pltpu.sync_copy(s0_hbm.at[pl.ds(base, CHT)], idx_v)
        pltpu.async_copy(rows_v, buf_hbm.at[idx_v], sem).wait()
        pltpu.sync_copy(s1_hbm.at[pl.ds(base, CHT)], idx_v)
        pltpu.async_copy(rows_v, buf_hbm.at[idx_v], sem).wait()

    return scat(xn, sidx0, sidx1)


def _ffn_body(cnt_ref, buf_ref, w13_ref, w2_ref, eo_ref):
    e = pl.program_id(0)
    cnt = cnt_ref[0, e]
    w13b = w13_ref[0].astype(jnp.bfloat16)
    w2b = w2_ref[0].astype(jnp.bfloat16)
    for sub in range(CAP // BLK):
        base = sub * BLK

        @pl.when((sub == 0) | (base < cnt))
        def _(base=base):
            ridx = lax.broadcasted_iota(jnp.int32, (BLK, 1), 0) + base
            x = jnp.where(ridx < cnt, _unpack2(buf_ref[pl.ds(base, BLK), :]),
                          jnp.bfloat16(0))
            gu = lax.dot_general(x, w13b, (((1,), (0,)), ((), ())),
                                 preferred_element_type=jnp.float32)
            g = gu[:, :I_MOE]
            u = gu[:, I_MOE:]
            a = (g * lax.logistic(g) * u).astype(jnp.bfloat16)
            eo_ref[pl.ds(base, BLK), :] = _pack2(
                lax.dot_general(a, w2b, (((1,), (0,)), ((), ())),
                                preferred_element_type=jnp.float32))


def _expert_ffn(cnt, buf, w1w3, w2):
    return pl.pallas_call(
        _ffn_body,
        grid=(E,),
        in_specs=[
            pl.BlockSpec(memory_space=pltpu.SMEM),
            pl.BlockSpec((CAP, PK), lambda e: (e, 0)),
            pl.BlockSpec((1, H, 2 * I_MOE), lambda e: (e, 0, 0)),
            pl.BlockSpec((1, I_MOE, H), lambda e: (e, 0, 0)),
        ],
        out_specs=pl.BlockSpec((CAP, PK), lambda e: (e, 0)),
        out_shape=jax.ShapeDtypeStruct((E * CAP, PK), jnp.int32),
    )(cnt, buf, w1w3, w2)


def _combine_gather(eo, cidx0, cidx1):
    mesh = plsc.VectorSubcoreMesh(core_axis_name="c", subcore_axis_name="s")

    @functools.partial(
        pl.kernel,
        mesh=mesh,
        out_type=(
            jax.ShapeDtypeStruct((T, PK), jnp.int32),
            jax.ShapeDtypeStruct((T, PK), jnp.int32),
        ),
        scratch_types=[
            pltpu.VMEM((CHT,), jnp.int32),
            pltpu.VMEM((CHT, PK), jnp.int32),
            pltpu.SemaphoreType.DMA,
        ],
    )
    def gath(eo_hbm, c0_hbm, c1_hbm, g0_hbm, g1_hbm, idx_v, rows_v, sem):
        wid = lax.axis_index("s") * 2 + lax.axis_index("c")
        base = wid * CHT
        pltpu.sync_copy(c0_hbm.at[pl.ds(base, CHT)], idx_v)
        pltpu.async_copy(eo_hbm.at[idx_v], rows_v, sem).wait()
        pltpu.sync_copy(rows_v, g0_hbm.at[pl.ds(base, CHT)])
        pltpu.sync_copy(c1_hbm.at[pl.ds(base, CHT)], idx_v)
        pltpu.async_copy(eo_hbm.at[idx_v], rows_v, sem).wait()
        pltpu.sync_copy(rows_v, g1_hbm.at[pl.ds(base, CHT)])

    return gath(eo, cidx0, cidx1)


def _shared_body(xn_ref, sg_ref, su_ref, sd_ref, sh_ref):
    sgb = sg_ref[...].astype(jnp.bfloat16)
    sub = su_ref[...].astype(jnp.bfloat16)
    sdb = sd_ref[...].astype(jnp.bfloat16)
    SB = 512
    for s in range(T // SB):
        xb = _unpack2(xn_ref[pl.ds(s * SB, SB), :])
        g = lax.dot_general(xb, sgb, (((1,), (0,)), ((), ())),
                            preferred_element_type=jnp.float32)
        u = lax.dot_general(xb, sub, (((1,), (0,)), ((), ())),
                            preferred_element_type=jnp.float32)
        a = (g * lax.logistic(g) * u).astype(jnp.bfloat16)
        sh_ref[pl.ds(s * SB, SB), :] = lax.dot_general(
            a, sdb, (((1,), (0,)), ((), ())),
            preferred_element_type=jnp.float32).astype(jnp.bfloat16)


def _shared_ffn(xn, sg, su, sd):
    i_sh = sg.shape[1]
    return pl.pallas_call(
        _shared_body,
        in_specs=[
            pl.BlockSpec((T, PK), lambda: (0, 0)),
            pl.BlockSpec((H, i_sh), lambda: (0, 0)),
            pl.BlockSpec((H, i_sh), lambda: (0, 0)),
            pl.BlockSpec((i_sh, H), lambda: (0, 0)),
        ],
        out_specs=pl.BlockSpec((T, H), lambda: (0, 0)),
        out_shape=jax.ShapeDtypeStruct((T, H), jnp.bfloat16),
    )(xn, sg, su, sd)


def _combine_body(sh_ref, g0_ref, g1_ref, wk0_ref, wk1_ref, hid_ref, out_ref):
    out_ref[...] = (sh_ref[...].astype(jnp.float32)
                    + _unpack2(g0_ref[...]).astype(jnp.float32) * wk0_ref[...]
                    + _unpack2(g1_ref[...]).astype(jnp.float32) * wk1_ref[...]
                    + hid_ref[...])


def _combine(sh, g0, g1, wk0, wk1, hid):
    n = T // TB
    return pl.pallas_call(
        _combine_body,
        grid=(n,),
        in_specs=[
            pl.BlockSpec((TB, H), lambda i: (i, 0)),
            pl.BlockSpec((TB, PK), lambda i: (i, 0)),
            pl.BlockSpec((TB, PK), lambda i: (i, 0)),
            pl.BlockSpec((TB, 1), lambda i: (i, 0)),
            pl.BlockSpec((TB, 1), lambda i: (i, 0)),
            pl.BlockSpec((TB, H), lambda i: (i, 0)),
        ],
        out_specs=pl.BlockSpec((TB, H), lambda i: (i, 0)),
        out_shape=jax.ShapeDtypeStruct((T, H), jnp.float32),
    )(sh, g0, g1, wk0, wk1, hid)


def kernel(hidden_states, gate_weight, w1w3, w2, shared_gate, shared_up,
           shared_down, ln_weight):
    x = hidden_states.reshape(T, H)
    (xn, wk0, wk1, cidx0, cidx1, sidx0, sidx1, cnt) = _router(
        x, gate_weight, ln_weight)
    sh = _shared_ffn(xn, shared_gate, shared_up, shared_down)
    buf = _moe_scatter(xn, sidx0.reshape(T), sidx1.reshape(T))
    eo = _expert_ffn(cnt, buf, w1w3, w2)
    g0, g1 = _combine_gather(eo, cidx0.reshape(T), cidx1.reshape(T))
    out = _combine(sh, g0, g1, wk0, wk1, x)
    return out.reshape(B, S, H)
